# baseline (speedup 1.0000x reference)
import functools

import jax
import jax.numpy as jnp
from jax import lax
from jax.experimental import pallas as pl
from jax.experimental.pallas import tpu as pltpu
from jax.experimental.pallas import tpu_sc as plsc

_BLOCK_S = 2048


def _pe_add_body(x_ref, pe_ref, o_ref):
    o_ref[...] = x_ref[...] + pe_ref[...][None, :, :]


def _kernel_tc(x, pos_emb):
    b, s, d = x.shape
    bs = _BLOCK_S if s % _BLOCK_S == 0 else s
    grid = (s // bs, b)
    return pl.pallas_call(
        _pe_add_body,
        grid=grid,
        in_specs=[
            pl.BlockSpec((1, bs, d), lambda i, j: (j, i, 0)),
            pl.BlockSpec((bs, d), lambda i, j: (i, 0)),
        ],
        out_specs=pl.BlockSpec((1, bs, d), lambda i, j: (j, i, 0)),
        out_shape=jax.ShapeDtypeStruct((b, s, d), x.dtype),
        compiler_params=pltpu.CompilerParams(
            dimension_semantics=("parallel", "parallel"),
        ),
    )(x, pos_emb)


def _kernel_sc(x, pos_emb):
    b, s, d = x.shape
    info = plsc.get_sparse_core_info()
    nc, ns = info.num_cores, info.num_subcores
    nw = nc * ns
    rows_w = s // nw
    chunk = 32
    n_chunks = rows_w // chunk
    nvec = d // 16
    mesh = plsc.VectorSubcoreMesh(core_axis_name="c", subcore_axis_name="s")

    @functools.partial(
        pl.kernel,
        mesh=mesh,
        out_type=jax.ShapeDtypeStruct((b, s, d), x.dtype),
        scratch_types=[
            pltpu.VMEM((chunk, d), jnp.float32),
            pltpu.VMEM((chunk, d), jnp.float32),
        ],
    )
    def k(x_hbm, pos_hbm, out_hbm, pos_v, x_v):
        wid = lax.axis_index("s") * nc + lax.axis_index("c")
        base = wid * rows_w

        def chunk_body(c, carry):
            row0 = base + c * chunk
            pltpu.sync_copy(pos_hbm.at[pl.ds(row0, chunk)], pos_v)
            for bi in range(b):
                pltpu.sync_copy(x_hbm.at[bi, pl.ds(row0, chunk)], x_v)

                def row_body(i, rcarry):
                    for kk in range(nvec):
                        sl = pl.ds(kk * 16, 16)
                        plsc.addupdate(x_v.at[i, sl], pos_v[i, sl])
                    return rcarry

                lax.fori_loop(0, chunk, row_body, 0)
                pltpu.sync_copy(x_v, out_hbm.at[bi, pl.ds(row0, chunk)])
            return carry

        lax.fori_loop(0, n_chunks, chunk_body, 0)

    return k(x, pos_emb)


def kernel(x, pos_emb):
    return _kernel_sc(x, pos_emb)
